# baseline (speedup 1.0000x reference)
import functools

import jax
import jax.numpy as jnp
from jax.experimental import pallas as pl
from jax.experimental.pallas import tpu as pltpu

_SIZES = {"FILE": 10000, "CLASS": 50000, "FUNCTION": 100000}
_D = 128
_RELS = [
    ("FILE", "OWNER", "CLASS", 0),
    ("FILE", "OWNER", "FUNCTION", 0),
    ("FILE", "CALL", "FUNCTION", 1),
    ("FILE", "IMPORT", "FILE", 1),
    ("FILE", "IMPORT", "CLASS", 1),
    ("FILE", "IMPORT", "FUNCTION", 1),
    ("CLASS", "OWNER", "CLASS", 2),
    ("CLASS", "OWNER", "FUNCTION", 2),
    ("FUNCTION", "OWNER", "CLASS", 2),
    ("FUNCTION", "OWNER", "FUNCTION", 2),
    ("CLASS", "CALL", "FUNCTION", 3),
    ("CLASS", "INHERITED", "CLASS", 3),
    ("FUNCTION", "CALL", "FUNCTION", 3),
]

_BN = 2000


def _mm_kernel(x_ref, w_ref, o_ref):
    o_ref[...] = jnp.dot(x_ref[...], w_ref[...],
                         preferred_element_type=jnp.float32)


def _matmul(x, w):
    n = x.shape[0]
    return pl.pallas_call(
        _mm_kernel,
        grid=(n // _BN,),
        in_specs=[
            pl.BlockSpec((_BN, _D), lambda i: (i, 0)),
            pl.BlockSpec((_D, _D), lambda i: (0, 0)),
        ],
        out_specs=pl.BlockSpec((_BN, _D), lambda i: (i, 0)),
        out_shape=jax.ShapeDtypeStruct((n, _D), jnp.float32),
    )(x, w)


def _combine_kernel(inv_r, acc_ref, x_ref, w_ref, b_ref, o_ref):
    dense = jnp.dot(x_ref[...], w_ref[...], preferred_element_type=jnp.float32)
    o_ref[...] = jnp.maximum(acc_ref[...] * inv_r + dense + b_ref[...], 0.0)


def _combine(acc, x, w, b, num_rels):
    n = x.shape[0]
    return pl.pallas_call(
        functools.partial(_combine_kernel, 1.0 / num_rels),
        grid=(n // _BN,),
        in_specs=[
            pl.BlockSpec((_BN, _D), lambda i: (i, 0)),
            pl.BlockSpec((_BN, _D), lambda i: (i, 0)),
            pl.BlockSpec((_D, _D), lambda i: (0, 0)),
            pl.BlockSpec((1, _D), lambda i: (0, 0)),
        ],
        out_specs=pl.BlockSpec((_BN, _D), lambda i: (i, 0)),
        out_shape=jax.ShapeDtypeStruct((n, _D), jnp.float32),
    )(acc, x, w, b)


def kernel(x_FILE, x_CLASS, x_FUNCTION, ei_FILE_OWNER_CLASS, ei_FILE_OWNER_FUNCTION, ei_FILE_CALL_FUNCTION, ei_FILE_IMPORT_FILE, ei_FILE_IMPORT_CLASS, ei_FILE_IMPORT_FUNCTION, ei_CLASS_OWNER_CLASS, ei_CLASS_OWNER_FUNCTION, ei_FUNCTION_OWNER_CLASS, ei_FUNCTION_OWNER_FUNCTION, ei_CLASS_CALL_FUNCTION, ei_CLASS_INHERITED_CLASS, ei_FUNCTION_CALL_FUNCTION, Wl, bl, Wr):
    x = {"FILE": x_FILE, "CLASS": x_CLASS, "FUNCTION": x_FUNCTION}
    ei = {
        ("FILE", "OWNER", "CLASS"): ei_FILE_OWNER_CLASS,
        ("FILE", "OWNER", "FUNCTION"): ei_FILE_OWNER_FUNCTION,
        ("FILE", "CALL", "FUNCTION"): ei_FILE_CALL_FUNCTION,
        ("FILE", "IMPORT", "FILE"): ei_FILE_IMPORT_FILE,
        ("FILE", "IMPORT", "CLASS"): ei_FILE_IMPORT_CLASS,
        ("FILE", "IMPORT", "FUNCTION"): ei_FILE_IMPORT_FUNCTION,
        ("CLASS", "OWNER", "CLASS"): ei_CLASS_OWNER_CLASS,
        ("CLASS", "OWNER", "FUNCTION"): ei_CLASS_OWNER_FUNCTION,
        ("FUNCTION", "OWNER", "CLASS"): ei_FUNCTION_OWNER_CLASS,
        ("FUNCTION", "OWNER", "FUNCTION"): ei_FUNCTION_OWNER_FUNCTION,
        ("CLASS", "CALL", "FUNCTION"): ei_CLASS_CALL_FUNCTION,
        ("CLASS", "INHERITED", "CLASS"): ei_CLASS_INHERITED_CLASS,
        ("FUNCTION", "CALL", "FUNCTION"): ei_FUNCTION_CALL_FUNCTION,
    }

    Wl1, bl1, Wr1 = Wl[1], bl[1], Wr[1]

    pairs = sorted({(s, ci) for (s, _, _, ci) in _RELS})
    y = {(s, ci): _matmul(x[s], Wl1[ci]) for (s, ci) in pairs}

    acc = {nt: jnp.zeros((_SIZES[nt], _D), jnp.float32) for nt in _SIZES}
    rels_per_dst = {nt: 0 for nt in _SIZES}
    for (s, r, d, ci) in _RELS:
        e = ei[(s, r, d)]
        msg = jnp.take(y[(s, ci)], e[0], axis=0)
        seg = jax.ops.segment_sum(msg, e[1], num_segments=_SIZES[d])
        cnt = jax.ops.segment_sum(jnp.ones((e.shape[1],), jnp.float32), e[1],
                                  num_segments=_SIZES[d])
        acc[d] = acc[d] + seg / jnp.maximum(cnt, 1.0)[:, None]
        rels_per_dst[d] += 1

    out = {}
    for nt in _SIZES:
        cis = [ci for (_, _, d, ci) in _RELS if d == nt]
        w_eff = sum(Wr1[ci] for ci in cis) / len(cis)
        b_eff = (sum(bl1[ci] for ci in cis) / len(cis)).reshape(1, _D)
        out[nt] = _combine(acc[nt], x[nt], w_eff, b_eff, rels_per_dst[nt])
    return (out["FILE"], out["CLASS"], out["FUNCTION"])
